# baseline (speedup 1.0000x reference)
import functools

import jax
import jax.numpy as jnp
from jax import lax
from jax.experimental import pallas as pl
from jax.experimental.pallas import tpu as pltpu
from jax.experimental.pallas import tpu_sc as plsc

_TOKENS = 32768
_D_MODEL = 4096
_NUM_EXPERTS = 64
_TOP_K = 8
_BT = 1024
_CH = 128
_NW = 32
_TPW = _TOKENS // _NW


def _tc_body(x_ref, w_ref, logits_ref, probs_ref, probs3d_ref):
    l = jnp.dot(x_ref[...], w_ref[...], preferred_element_type=jnp.float32)
    logits_ref[...] = l
    for c in range(_BT // _CH):
        rows = pl.ds(c * _CH, _CH)
        lt = logits_ref[rows, :].T
        m0 = jnp.max(lt, axis=0, keepdims=True)
        ex = jnp.exp(lt - jnp.broadcast_to(m0, (_NUM_EXPERTS, _CH)))
        s = jnp.sum(ex, axis=0, keepdims=True)
        rs = 1.0 / s
        pt = ex * jnp.broadcast_to(rs, (_NUM_EXPERTS, _CH))
        probs_ref[rows, :] = pt.T
        probs3d_ref[0, :, pl.ds(c * _CH, _CH)] = pt


def _tc_call(x, W):
    grid = (_TOKENS // _BT,)
    out_shapes = (
        jax.ShapeDtypeStruct((_TOKENS, _NUM_EXPERTS), jnp.float32),
        jax.ShapeDtypeStruct((_TOKENS, _NUM_EXPERTS), jnp.float32),
        jax.ShapeDtypeStruct((_NW, _NUM_EXPERTS, _TPW), jnp.float32),
    )
    return pl.pallas_call(
        _tc_body,
        grid=grid,
        in_specs=[
            pl.BlockSpec((_BT, _D_MODEL), lambda i: (i, 0)),
            pl.BlockSpec((_D_MODEL, _NUM_EXPERTS), lambda i: (0, 0)),
        ],
        out_specs=(
            pl.BlockSpec((_BT, _NUM_EXPERTS), lambda i: (i, 0)),
            pl.BlockSpec((_BT, _NUM_EXPERTS), lambda i: (i, 0)),
            pl.BlockSpec((1, _NUM_EXPERTS, _BT), lambda i: (i, 0, 0)),
        ),
        out_shape=out_shapes,
        compiler_params=pltpu.CompilerParams(
            dimension_semantics=("arbitrary",),
        ),
    )(x, W)


def _sc_topk_kernel(probs3d):
    mesh = plsc.VectorSubcoreMesh(core_axis_name="c", subcore_axis_name="s")

    @functools.partial(
        pl.kernel,
        mesh=mesh,
        out_type=[
            jax.ShapeDtypeStruct((_NW, _TOP_K, _TPW), jnp.float32),
            jax.ShapeDtypeStruct((_NW, _TOP_K, _TPW), jnp.int32),
        ],
        scratch_types=[
            pltpu.VMEM((_NUM_EXPERTS, _TPW), jnp.float32),
            pltpu.VMEM((_TOP_K, _TPW), jnp.float32),
            pltpu.VMEM((_TOP_K, _TPW), jnp.int32),
        ],
        compiler_params=pltpu.CompilerParams(needs_layout_passes=False),
    )
    def sc_topk(probs3d_hbm, topw_hbm, topi_hbm, buf, wbuf, ibuf):
        cc = lax.axis_index("c")
        ss = lax.axis_index("s")
        w = ss * 2 + cc
        pltpu.sync_copy(probs3d_hbm.at[w], buf)

        def group(g, carry):
            base = g * 16
            tok = base + lax.iota(jnp.int32, 16)
            neg = jnp.full((16,), -1.0, jnp.float32)
            for j in range(_TOP_K):
                m = jnp.full((16,), -1.0, jnp.float32)
                mi = jnp.zeros((16,), jnp.int32)
                for e in range(_NUM_EXPERTS):
                    v = buf[e, pl.ds(base, 16)]
                    gt = v > m
                    m = jnp.where(gt, v, m)
                    mi = jnp.where(gt, e, mi)
                wbuf[j, pl.ds(base, 16)] = m
                ibuf[j, pl.ds(base, 16)] = mi
                plsc.store_scatter(buf, [mi, tok], neg)
            return carry

        lax.fori_loop(0, _TPW // 16, group, 0)
        pltpu.sync_copy(wbuf, topw_hbm.at[w])
        pltpu.sync_copy(ibuf, topi_hbm.at[w])

    return sc_topk(probs3d)


@jax.jit
def kernel(x, W):
    logits, probs, probs3d = _tc_call(x, W)
    topw3d, topi3d = _sc_topk_kernel(probs3d)
    topw = jnp.transpose(topw3d, (0, 2, 1)).reshape(_TOKENS, _TOP_K)
    topi = jnp.transpose(topi3d, (0, 2, 1)).reshape(_TOKENS, _TOP_K)
    return logits, probs, topw, topi


# speedup vs baseline: 1.2177x; 1.2177x over previous
import jax
import jax.numpy as jnp
from jax.experimental import pallas as pl
from jax.experimental.pallas import tpu as pltpu

_TOKENS = 32768
_D_MODEL = 4096
_NUM_EXPERTS = 64
_TOP_K = 8
_BT = 1024
_CH = 128


def _router_body(x_ref, w_ref, logits_ref, probs_ref, topw_ref, topi_ref):
    l = jnp.dot(x_ref[...], w_ref[...], preferred_element_type=jnp.float32)
    logits_ref[...] = l

    invrows = jax.lax.broadcasted_iota(jnp.int32, (_NUM_EXPERTS, _CH), 0)
    invrows = (_NUM_EXPERTS - 1) - invrows
    for c in range(_BT // _CH):
        rows = pl.ds(c * _CH, _CH)
        lt = logits_ref[rows, :].T

        m0 = jnp.max(lt, axis=0, keepdims=True)
        m0b = jnp.broadcast_to(m0, (_NUM_EXPERTS, _CH))
        ex = jnp.exp(lt - m0b)
        s = jnp.sum(ex, axis=0, keepdims=True)
        rs = 1.0 / s
        probs_ref[rows, :] = (ex * jnp.broadcast_to(rs, (_NUM_EXPERTS, _CH))).T

        v = lt
        ls = []
        ids = []
        for j in range(_TOP_K):
            mj = m0 if j == 0 else jnp.max(v, axis=0, keepdims=True)
            mjb = jnp.broadcast_to(mj, (_NUM_EXPERTS, _CH))
            t = jnp.where(v == mjb, invrows, -1)
            am = jnp.max(t, axis=0, keepdims=True)
            ls.append(mj)
            ids.append((_NUM_EXPERTS - 1) - am)
            v = jnp.where(t == jnp.broadcast_to(am, (_NUM_EXPERTS, _CH)), -jnp.inf, v)
        lsel = jnp.concatenate(ls, axis=0)
        cols = pl.ds(c * _CH, _CH)
        topw_ref[:, cols] = jnp.exp(lsel - jnp.broadcast_to(m0, (_TOP_K, _CH))) * (
            jnp.broadcast_to(rs, (_TOP_K, _CH))
        )
        topi_ref[:, cols] = jnp.concatenate(ids, axis=0)


@jax.jit
def kernel(x, W):
    grid = (_TOKENS // _BT,)
    out_shapes = (
        jax.ShapeDtypeStruct((_TOKENS, _NUM_EXPERTS), jnp.float32),
        jax.ShapeDtypeStruct((_TOKENS, _NUM_EXPERTS), jnp.float32),
        jax.ShapeDtypeStruct((_TOP_K, _TOKENS), jnp.float32),
        jax.ShapeDtypeStruct((_TOP_K, _TOKENS), jnp.int32),
    )
    logits, probs, topw_t, topi_t = pl.pallas_call(
        _router_body,
        grid=grid,
        in_specs=[
            pl.BlockSpec((_BT, _D_MODEL), lambda i: (i, 0)),
            pl.BlockSpec((_D_MODEL, _NUM_EXPERTS), lambda i: (0, 0)),
        ],
        out_specs=(
            pl.BlockSpec((_BT, _NUM_EXPERTS), lambda i: (i, 0)),
            pl.BlockSpec((_BT, _NUM_EXPERTS), lambda i: (i, 0)),
            pl.BlockSpec((_TOP_K, _BT), lambda i: (0, i)),
            pl.BlockSpec((_TOP_K, _BT), lambda i: (0, i)),
        ),
        out_shape=out_shapes,
        compiler_params=pltpu.CompilerParams(
            dimension_semantics=("arbitrary",),
        ),
    )(x, W)
    return logits, probs, topw_t.T, topi_t.T
